# Initial kernel scaffold; baseline (speedup 1.0000x reference)
import jax
import jax.numpy as jnp
from jax.experimental import pallas as pl


def kernel(x, emb, W, b):
    raise NotImplementedError("write your pallas kernel here")


# speedup vs baseline: 4.8596x; 4.8596x over previous
import functools

import jax
import jax.numpy as jnp
from jax.experimental import pallas as pl

_VOCAB_PAD = 512
_BN = 2048


def _table_body(emb_ref, w_ref, b_ref, t_ref):
    prod = jax.lax.dot_general(
        emb_ref[...], w_ref[...],
        dimension_numbers=(((1,), (1,)), ((), ())),
        preferred_element_type=jnp.float32,
    )
    t_ref[...] = jnp.tanh(prod + b_ref[...])


def _lookup_body(x_ref, t_ref, o_ref):
    idx = x_ref[...]
    iota = jax.lax.broadcasted_iota(jnp.int32, (_BN, _VOCAB_PAD), 1)
    onehot = (idx == iota).astype(jnp.float32)
    o_ref[...] = jax.lax.dot_general(
        onehot, t_ref[...],
        dimension_numbers=(((1,), (0,)), ((), ())),
        preferred_element_type=jnp.float32,
    )


@jax.jit
def kernel(x, emb, W, b):
    B, L = x.shape
    V, D = emb.shape
    O = W.shape[0]
    emb_pad = jnp.zeros((_VOCAB_PAD, D), jnp.float32).at[:V].set(emb)

    table = pl.pallas_call(
        _table_body,
        out_shape=jax.ShapeDtypeStruct((_VOCAB_PAD, O), jnp.float32),
    )(emb_pad, W, b.reshape(1, O))

    n = B * L
    x_col = x.reshape(n, 1).astype(jnp.int32)
    out = pl.pallas_call(
        _lookup_body,
        grid=(n // _BN,),
        in_specs=[
            pl.BlockSpec((_BN, 1), lambda i: (i, 0)),
            pl.BlockSpec((_VOCAB_PAD, O), lambda i: (0, 0)),
        ],
        out_specs=pl.BlockSpec((_BN, O), lambda i: (i, 0)),
        out_shape=jax.ShapeDtypeStruct((n, O), jnp.float32),
    )(x_col, table)
    return out.reshape(B, L, O)
